# Initial kernel scaffold; baseline (speedup 1.0000x reference)
import jax
import jax.numpy as jnp
from jax.experimental import pallas as pl


def kernel(x, pos_embedding):
    raise NotImplementedError("write your pallas kernel here")


# speedup vs baseline: 1.9639x; 1.9639x over previous
import jax
import jax.numpy as jnp
from jax.experimental import pallas as pl


def _add_kernel(x_ref, p_ref, o_ref):
    o_ref[...] = x_ref[...] + p_ref[...][None, :, :]


def kernel(x, pos_embedding):
    B, S, D = x.shape
    SBLK = 512
    return pl.pallas_call(
        _add_kernel,
        grid=(S // SBLK,),
        in_specs=[
            pl.BlockSpec((B, SBLK, D), lambda s: (0, s, 0)),
            pl.BlockSpec((SBLK, D), lambda s: (s, 0)),
        ],
        out_specs=pl.BlockSpec((B, SBLK, D), lambda s: (0, s, 0)),
        out_shape=jax.ShapeDtypeStruct((B, S, D), x.dtype),
    )(x, pos_embedding)
